# Initial kernel scaffold; baseline (speedup 1.0000x reference)
import functools

import jax
import jax.numpy as jnp
from jax import lax
from jax.experimental import pallas as pl
from jax.experimental.pallas import tpu as pltpu
from jax.experimental.pallas import tpu_sc as plsc


def _gather_kernel(B, S, D):
    mesh = plsc.VectorSubcoreMesh(core_axis_name="c", subcore_axis_name="s")

    @functools.partial(
        pl.kernel,
        mesh=mesh,
        out_type=jax.ShapeDtypeStruct((B * 4, D), jnp.float32),
        scratch_types=[
            pltpu.VMEM((16,), jnp.int32),
            pltpu.VMEM((B * 4, D), jnp.float32),
            pltpu.SemaphoreType.DMA,
        ],
    )
    def k(x_hbm, ann_hbm, out_hbm, idx_v, rows_v, sem):
        wid = lax.axis_index("s") * 2 + lax.axis_index("c")

        @pl.when(wid == 0)
        def _():
            pltpu.sync_copy(ann_hbm, idx_v)
            lane = lax.iota(jnp.int32, (16,))
            idx_v[...] = idx_v[...] + (lane >> 2) * S
            pltpu.async_copy(x_hbm.at[idx_v], rows_v, sem).wait()
            pltpu.sync_copy(rows_v, out_hbm)

    return k


def kernel(x, src_tokens, annotation):
    B, S, D = x.shape
    x_flat = x.reshape(B * S, D)
    ann = annotation.reshape(-1).astype(jnp.int32)
    out = _gather_kernel(B, S, D)(x_flat, ann)
    return out.reshape(B, 4 * D)


# speedup vs baseline: 1.0677x; 1.0677x over previous
import functools

import jax
import jax.numpy as jnp
from jax import lax
from jax.experimental import pallas as pl
from jax.experimental.pallas import tpu as pltpu
from jax.experimental.pallas import tpu_sc as plsc


def _gather_kernel(B, S, D):
    mesh = plsc.VectorSubcoreMesh(core_axis_name="c", subcore_axis_name="s")

    @functools.partial(
        pl.kernel,
        mesh=mesh,
        out_type=jax.ShapeDtypeStruct((B * 4, D), jnp.float32),
        scratch_types=[
            pltpu.VMEM((16,), jnp.int32),
            pltpu.VMEM((B * 4, D), jnp.float32),
            pltpu.SemaphoreType.DMA,
        ],
    )
    def k(x_hbm, ann_hbm, out_hbm, idx_v, rows_v, sem):
        wid = lax.axis_index("s") * 2 + lax.axis_index("c")

        @pl.when(wid == 0)
        def _():
            pltpu.sync_copy(ann_hbm, idx_v)
            lane = lax.iota(jnp.int32, 16)
            idx_v[...] = idx_v[...] + (lane >> 2) * S
            pltpu.async_copy(x_hbm.at[idx_v], rows_v, sem).wait()
            pltpu.sync_copy(rows_v, out_hbm)

    return k


def kernel(x, src_tokens, annotation):
    B, S, D = x.shape
    x_flat = x.reshape(B * S, D)
    ann = annotation.reshape(-1).astype(jnp.int32)
    out = _gather_kernel(B, S, D)(x_flat, ann)
    return out.reshape(B, 4 * D)


# speedup vs baseline: 1.1126x; 1.0421x over previous
import functools

import jax
import jax.numpy as jnp
from jax import lax
from jax.experimental import pallas as pl
from jax.experimental.pallas import tpu as pltpu
from jax.experimental.pallas import tpu_sc as plsc


def _gather_kernel(B, S, D):
    mesh = plsc.VectorSubcoreMesh(core_axis_name="c", subcore_axis_name="s")

    @functools.partial(
        pl.kernel,
        mesh=mesh,
        out_type=jax.ShapeDtypeStruct((B * 4, D), jnp.float32),
        scratch_types=[
            pltpu.VMEM((16,), jnp.int32),
            pltpu.VMEM((1, D), jnp.float32),
            pltpu.SemaphoreType.DMA,
        ],
    )
    def k(x_hbm, ann_hbm, out_hbm, idx_v, row_v, sem):
        wid = lax.axis_index("s") * 2 + lax.axis_index("c")

        @pl.when(wid < B * 4)
        def _():
            pltpu.sync_copy(ann_hbm, idx_v)
            lane = lax.iota(jnp.int32, 16)
            rows = idx_v[...] + (lane >> 2) * S
            perm = (lane + wid) & 15
            dnums = lax.GatherDimensionNumbers(
                offset_dims=(), collapsed_slice_dims=(0,), start_index_map=(0,))
            idx_v[...] = lax.gather(
                rows, perm.reshape(16, 1), dnums, (1,),
                mode=lax.GatherScatterMode.PROMISE_IN_BOUNDS)
            pltpu.async_copy(x_hbm.at[idx_v.at[pl.ds(0, 1)]], row_v, sem).wait()
            pltpu.sync_copy(row_v, out_hbm.at[pl.ds(wid, 1)])

    return k


def kernel(x, src_tokens, annotation):
    B, S, D = x.shape
    x_flat = x.reshape(B * S, D)
    ann = annotation.reshape(-1).astype(jnp.int32)
    out = _gather_kernel(B, S, D)(x_flat, ann)
    return out.reshape(B, 4 * D)


# speedup vs baseline: 1.1781x; 1.0588x over previous
import functools

import jax
import jax.numpy as jnp
from jax import lax
from jax.experimental import pallas as pl
from jax.experimental.pallas import tpu as pltpu
from jax.experimental.pallas import tpu_sc as plsc


def _gather_kernel(B, S, D):
    mesh = plsc.VectorSubcoreMesh(
        core_axis_name="c", subcore_axis_name="s", num_cores=1)

    @functools.partial(
        pl.kernel,
        mesh=mesh,
        out_type=jax.ShapeDtypeStruct((B * 4, D), jnp.float32),
        scratch_types=[
            pltpu.VMEM((16,), jnp.int32),
            pltpu.VMEM((1, D), jnp.float32),
            pltpu.SemaphoreType.DMA,
        ],
    )
    def k(x_hbm, ann_hbm, out_hbm, idx_v, row_v, sem):
        wid = lax.axis_index("s") + lax.axis_index("c")

        @pl.when(wid < B * 4)
        def _():
            pltpu.sync_copy(ann_hbm, idx_v)
            lane = lax.iota(jnp.int32, 16)
            rows = idx_v[...] + (lane >> 2) * S
            perm = (lane + wid) & 15
            dnums = lax.GatherDimensionNumbers(
                offset_dims=(), collapsed_slice_dims=(0,), start_index_map=(0,))
            idx_v[...] = lax.gather(
                rows, perm.reshape(16, 1), dnums, (1,),
                mode=lax.GatherScatterMode.PROMISE_IN_BOUNDS)
            pltpu.async_copy(x_hbm.at[idx_v.at[pl.ds(0, 1)]], row_v, sem).wait()
            pltpu.sync_copy(row_v, out_hbm.at[pl.ds(wid, 1)])

    return k


def kernel(x, src_tokens, annotation):
    B, S, D = x.shape
    x_flat = x.reshape(B * S, D)
    ann = annotation.reshape(-1).astype(jnp.int32)
    out = _gather_kernel(B, S, D)(x_flat, ann)
    return out.reshape(B, 4 * D)
